# baseline (speedup 1.0000x reference)
import jax
import jax.numpy as jnp
from jax import lax
from jax.experimental import pallas as pl
from jax.experimental.pallas import tpu as pltpu
from jax.experimental.pallas import tpu_sc as plsc

_DIM = 8192
_B = 32
_K = 8
_TILE = 960
_NT = -(-_DIM // _TILE)

_NEG_INF = float("-inf")

_L = 16
_NC = 2
_ROWS_PER_W = 64
_CHUNK_ROWS = 4


def _fused_kernel(x_ref, w_ref, vals_ref, idx_ref):
    t = pl.program_id(0)

    @pl.when(t == 0)
    def _init():
        vals_ref[...] = jnp.full((_B, _K), _NEG_INF, jnp.float32)
        idx_ref[...] = jnp.zeros((_B, _K), jnp.int32)

    y = jax.lax.dot_general(
        x_ref[...], w_ref[...],
        (((1,), (1,)), ((), ())),
        preferred_element_type=jnp.float32,
    )

    base = t * _TILE
    col = jax.lax.broadcasted_iota(jnp.int32, (_B, _TILE), 1) + base
    y = jnp.where(col < _DIM, y, _NEG_INF)

    cand_v = jnp.concatenate([vals_ref[...], y], axis=1)
    cand_i = jnp.concatenate([idx_ref[...], col], axis=1)
    pos = jax.lax.broadcasted_iota(jnp.int32, cand_v.shape, 1)

    new_v = []
    new_i = []
    for _ in range(_K):
        m = jnp.max(cand_v, axis=-1, keepdims=True)
        a = jnp.argmax(cand_v, axis=-1).astype(jnp.int32)[:, None]
        hit = pos == a
        sel_i = jnp.sum(jnp.where(hit, cand_i, 0), axis=-1, keepdims=True)
        new_v.append(m)
        new_i.append(sel_i)
        cand_v = jnp.where(hit, _NEG_INF, cand_v)

    vals_ref[...] = jnp.concatenate(new_v, axis=1)
    idx_ref[...] = jnp.concatenate(new_i, axis=1)


def _fused(x, W):
    return pl.pallas_call(
        _fused_kernel,
        grid=(_NT,),
        in_specs=[
            pl.BlockSpec((_B, _DIM), lambda i: (0, 0)),
            pl.BlockSpec((_TILE, _DIM), lambda i: (i, 0)),
        ],
        out_specs=[
            pl.BlockSpec((_B, _K), lambda i: (0, 0)),
            pl.BlockSpec((_B, _K), lambda i: (0, 0)),
        ],
        out_shape=[
            jax.ShapeDtypeStruct((_B, _K), jnp.float32),
            jax.ShapeDtypeStruct((_B, _K), jnp.int32),
        ],
        compiler_params=pltpu.CompilerParams(
            dimension_semantics=("arbitrary",),
            vmem_limit_bytes=128 * 1024 * 1024,
        ),
    )(x, W)


def _sc_reader_body(w_hbm, out_hbm, buf_v, sum_v):
    c = lax.axis_index("c")
    s = lax.axis_index("s")
    wid = s * _NC + c
    row0 = wid * _ROWS_PER_W

    sum_v[...] = jnp.zeros((_L,), jnp.float32)

    def body(i, acc):
        pltpu.sync_copy(
            w_hbm.at[pl.ds(row0 + i * _CHUNK_ROWS, _CHUNK_ROWS)], buf_v)
        return acc + buf_v[0, pl.ds(0, _L)]

    acc = lax.fori_loop(0, _ROWS_PER_W // _CHUNK_ROWS,
                        body, jnp.zeros((_L,), jnp.float32))
    sum_v[...] = acc
    pltpu.sync_copy(sum_v, out_hbm.at[wid])


def _sc_reader(W):
    mesh = plsc.VectorSubcoreMesh(core_axis_name="c", subcore_axis_name="s")
    f = pl.kernel(
        _sc_reader_body,
        out_type=jax.ShapeDtypeStruct((_B, _L), jnp.float32),
        mesh=mesh,
        scratch_types=[
            pltpu.VMEM((_CHUNK_ROWS, _DIM), jnp.float32),
            pltpu.VMEM((_L,), jnp.float32),
        ],
    )
    return f(W)


def kernel(x, W):
    s = _sc_reader(W)
    vals, idx = _fused(x, W)
    vals = vals + jnp.sum(s) * jnp.float32(0.0)
    return (vals, idx)


# speedup vs baseline: 1.4019x; 1.4019x over previous
import jax
import jax.numpy as jnp
from jax.experimental import pallas as pl
from jax.experimental.pallas import tpu as pltpu

_DIM = 8192
_B = 32
_K = 8
_TILE = 960
_NT = -(-_DIM // _TILE)

_NEG_INF = float("-inf")


def _fused_kernel(x_ref, w_ref, vals_ref, idx_ref):
    t = pl.program_id(0)

    @pl.when(t == 0)
    def _init():
        vals_ref[...] = jnp.full((_B, _K), _NEG_INF, jnp.float32)
        idx_ref[...] = jnp.zeros((_B, _K), jnp.int32)

    y = jax.lax.dot_general(
        x_ref[...], w_ref[...],
        (((1,), (1,)), ((), ())),
        preferred_element_type=jnp.float32,
    )

    base = t * _TILE
    col = jax.lax.broadcasted_iota(jnp.int32, (_B, _TILE), 1) + base
    y = jnp.where(col < _DIM, y, _NEG_INF)

    cand_v = jnp.concatenate([vals_ref[...], y], axis=1)
    cand_i = jnp.concatenate([idx_ref[...], col], axis=1)
    pos = jax.lax.broadcasted_iota(jnp.int32, cand_v.shape, 1)

    new_v = []
    new_i = []
    for _ in range(_K):
        m = jnp.max(cand_v, axis=-1, keepdims=True)
        a = jnp.argmax(cand_v, axis=-1).astype(jnp.int32)
        a = a[:, None]
        hit = pos == a
        sel_i = jnp.sum(jnp.where(hit, cand_i, 0), axis=-1, keepdims=True)
        new_v.append(m)
        new_i.append(sel_i)
        cand_v = jnp.where(hit, _NEG_INF, cand_v)

    vals_ref[...] = jnp.concatenate(new_v, axis=1)
    idx_ref[...] = jnp.concatenate(new_i, axis=1)


def kernel(x, W):
    vals, idx = pl.pallas_call(
        _fused_kernel,
        grid=(_NT,),
        in_specs=[
            pl.BlockSpec((_B, _DIM), lambda i: (0, 0)),
            pl.BlockSpec((_TILE, _DIM), lambda i: (i, 0)),
        ],
        out_specs=[
            pl.BlockSpec((_B, _K), lambda i: (0, 0)),
            pl.BlockSpec((_B, _K), lambda i: (0, 0)),
        ],
        out_shape=[
            jax.ShapeDtypeStruct((_B, _K), jnp.float32),
            jax.ShapeDtypeStruct((_B, _K), jnp.int32),
        ],
        compiler_params=pltpu.CompilerParams(
            dimension_semantics=("arbitrary",),
            vmem_limit_bytes=128 * 1024 * 1024,
        ),
    )(x, W)
    return (vals, idx)
